# Initial kernel scaffold; baseline (speedup 1.0000x reference)
import jax
import jax.numpy as jnp
from jax.experimental import pallas as pl


def kernel(edge_index, edge_type, h_index, t_index, r_index, query, rel_w, lin_w, lin_b, ln_scale, ln_bias, mlp_w1, mlp_b1, mlp_w2, mlp_b2):
    raise NotImplementedError("write your pallas kernel here")


# speedup vs baseline: 26.4502x; 26.4502x over previous
import jax
import jax.numpy as jnp
from jax import lax
from jax.experimental import pallas as pl
from jax.experimental.pallas import tpu as pltpu
from jax.experimental.pallas import tpu_sc as plsc

NN = 10000
NE = 160000
DD = 128
NB = 2
NTAIL = 34
EPS = 1e-5

NSUB = 16
EPT = 10240
EP = EPT * NSUB
CH = 128
NCHUNK = EPT // CH
ACC_ROWS = 10112

ROWS = 1000
NBLK = NN // ROWS
GRID = NB * NBLK


def _msg_body(hid, srcg, dst, et, rel, agg,
              acc, rel_sp, src_v, dst_v, et_v, hrow, rrow,
              sem_h, sem_r):
    c = lax.axis_index("c")
    s = lax.axis_index("s")
    z16 = jnp.zeros((16,), jnp.float32)

    def zrow(j, carry):
        for k in range(8):
            hrow[j, pl.ds(k * 16, 16)] = z16
        return carry
    lax.fori_loop(0, CH, zrow, 0)

    zb = s * 632
    for k in range(4):
        pltpu.sync_copy(hrow, acc.at[pl.ds(zb + k * 128, 128)])
    pltpu.sync_copy(hrow.at[pl.ds(0, 120)], acc.at[pl.ds(zb + 512, 120)])

    @pl.when(s == 0)
    def _():
        pltpu.sync_copy(rel, hrow.at[pl.ds(0, 104)])
        pltpu.sync_copy(hrow.at[pl.ds(0, 104)], rel_sp)

    plsc.subcore_barrier()

    def chunk(i, carry):
        eoff = s * EPT + i * CH
        pltpu.sync_copy(srcg.at[pl.ds(c * EP + eoff, CH)], src_v)
        pltpu.sync_copy(dst.at[pl.ds(eoff, CH)], dst_v)
        pltpu.sync_copy(et.at[pl.ds(eoff, CH)], et_v)
        cph = pltpu.async_copy(hid.at[src_v], hrow, sem_h)
        cpr = pltpu.async_copy(rel_sp.at[et_v], rrow, sem_r)
        cph.wait()
        cpr.wait()

        def mulrow(j, cc):
            for k in range(8):
                sl = pl.ds(k * 16, 16)
                hrow[j, sl] = hrow[j, sl] * rrow[j, sl]
            return cc
        lax.fori_loop(0, CH, mulrow, 0)

        pltpu.sync_copy(hrow, acc.at[dst_v], add=True)
        return carry
    lax.fori_loop(0, NCHUNK, chunk, 0)

    plsc.subcore_barrier()

    ob = s * 632

    @pl.when(s < 15)
    def _():
        for k in range(4):
            pltpu.sync_copy(acc.at[pl.ds(ob + k * 128, 128)], hrow)
            pltpu.sync_copy(hrow, agg.at[pl.ds(c * NN + ob + k * 128, 128)])
        pltpu.sync_copy(acc.at[pl.ds(ob + 512, 120)], hrow.at[pl.ds(0, 120)])
        pltpu.sync_copy(hrow.at[pl.ds(0, 120)],
                        agg.at[pl.ds(c * NN + ob + 512, 120)])

    @pl.when(s == 15)
    def _():
        for k in range(4):
            pltpu.sync_copy(acc.at[pl.ds(ob + k * 128, 128)], hrow)
            pltpu.sync_copy(hrow, agg.at[pl.ds(c * NN + ob + k * 128, 128)])
        pltpu.sync_copy(acc.at[pl.ds(ob + 512, 8)], hrow.at[pl.ds(0, 8)])
        pltpu.sync_copy(hrow.at[pl.ds(0, 8)],
                        agg.at[pl.ds(c * NN + ob + 512, 8)])


import functools


@functools.lru_cache(maxsize=None)
def _get_msg_kernel():
    return pl.kernel(
    _msg_body,
    out_type=jax.ShapeDtypeStruct((NB * NN, DD), jnp.float32),
    mesh=plsc.VectorSubcoreMesh(core_axis_name="c", subcore_axis_name="s", num_cores=2, num_subcores=16),
    scratch_types=[
        pltpu.VMEM_SHARED((ACC_ROWS, DD), jnp.float32),
        pltpu.VMEM_SHARED((104, DD), jnp.float32),
        pltpu.VMEM((CH,), jnp.int32),
        pltpu.VMEM((CH,), jnp.int32),
        pltpu.VMEM((CH,), jnp.int32),
        pltpu.VMEM((CH, DD), jnp.float32),
        pltpu.VMEM((CH, DD), jnp.float32),
        pltpu.SemaphoreType.DMA,
        pltpu.SemaphoreType.DMA,
    ],
    )


def _bound_body(hidx_ref, oh_ref, query_ref, bout_ref, qout_ref):
    i = pl.program_id(0)
    b = i // NBLK
    base_n = (i % NBLK) * ROWS
    q = jnp.dot(oh_ref[...], query_ref[...], preferred_element_type=jnp.float32)
    qout_ref[...] = q
    hi = jnp.where(b == 0, hidx_ref[0, 0], hidx_ref[0, 1])
    mask = (lax.broadcasted_iota(jnp.int32, (ROWS, 1), 0) + base_n == hi
            ).astype(jnp.float32)
    qsel = jnp.where(b == 0, q[0:1, :], q[1:2, :])
    bout_ref[...] = mask * qsel


def _bound_call(hidx, oh, qpad):
    return pl.pallas_call(
        _bound_body,
        grid=(GRID,),
        in_specs=[
            pl.BlockSpec(memory_space=pltpu.SMEM),
            pl.BlockSpec((8, 104), lambda i: (0, 0)),
            pl.BlockSpec((104, DD), lambda i: (0, 0)),
        ],
        out_specs=[
            pl.BlockSpec((ROWS, DD), lambda i: (i, 0)),
            pl.BlockSpec((8, DD), lambda i: (0, 0)),
        ],
        out_shape=[
            jax.ShapeDtypeStruct((NB * NN, DD), jnp.float32),
            jax.ShapeDtypeStruct((8, DD), jnp.float32),
        ],
    )(hidx, oh, qpad)


def _dense_body(hidx_ref, hid_ref, agg_ref, w1_ref, w2_ref, bb_ref, g_ref,
                be_ref, q_ref, out_ref):
    i = pl.program_id(0)
    b = i // NBLK
    base_n = (i % NBLK) * ROWS
    h = hid_ref[...]
    a = agg_ref[...]
    hi = jnp.where(b == 0, hidx_ref[0, 0], hidx_ref[0, 1])
    mask = (lax.broadcasted_iota(jnp.int32, (ROWS, 1), 0) + base_n == hi
            ).astype(jnp.float32)
    qsel = jnp.where(b == 0, q_ref[0:1, :], q_ref[1:2, :])
    a = a + mask * qsel
    out = (jnp.dot(h, w1_ref[...], preferred_element_type=jnp.float32)
           + jnp.dot(a, w2_ref[...], preferred_element_type=jnp.float32)
           + bb_ref[...])
    mean = jnp.mean(out, axis=-1, keepdims=True)
    var = jnp.mean((out - mean) * (out - mean), axis=-1, keepdims=True)
    out = (out - mean) / jnp.sqrt(var + EPS) * g_ref[...] + be_ref[...]
    out_ref[...] = jnp.maximum(out, 0.0) + h


def _dense_call(hid, aggr, w1, w2, bb, g, be, hidx, q):
    return pl.pallas_call(
        _dense_body,
        grid=(GRID,),
        in_specs=[
            pl.BlockSpec(memory_space=pltpu.SMEM),
            pl.BlockSpec((ROWS, DD), lambda i: (i, 0)),
            pl.BlockSpec((ROWS, DD), lambda i: (i, 0)),
            pl.BlockSpec((DD, DD), lambda i: (0, 0)),
            pl.BlockSpec((DD, DD), lambda i: (0, 0)),
            pl.BlockSpec((1, DD), lambda i: (0, 0)),
            pl.BlockSpec((1, DD), lambda i: (0, 0)),
            pl.BlockSpec((1, DD), lambda i: (0, 0)),
            pl.BlockSpec((8, DD), lambda i: (0, 0)),
        ],
        out_specs=pl.BlockSpec((ROWS, DD), lambda i: (i, 0)),
        out_shape=jax.ShapeDtypeStruct((NB * NN, DD), jnp.float32),
    )(hidx, hid, aggr, w1, w2, bb, g, be, q)


def _score_body(hid_ref, w1a_ref, w1b_ref, b1_ref, w2_ref, b2_ref, q_ref,
                out_ref):
    i = pl.program_id(0)
    b = i // NBLK
    h = hid_ref[...]
    qsel = jnp.where(b == 0, q_ref[0:1, :], q_ref[1:2, :])
    pre = (jnp.dot(qsel, w1b_ref[...], preferred_element_type=jnp.float32)
           + b1_ref[...])
    h1 = jnp.maximum(
        jnp.dot(h, w1a_ref[...], preferred_element_type=jnp.float32) + pre, 0.0)
    out_ref[...] = (jnp.dot(h1, w2_ref[...], preferred_element_type=jnp.float32)
                    + b2_ref[...])


def _score_call(hid, w1a, w1b, b1, w2p, b2p, q):
    return pl.pallas_call(
        _score_body,
        grid=(GRID,),
        in_specs=[
            pl.BlockSpec((ROWS, DD), lambda i: (i, 0)),
            pl.BlockSpec((DD, 2 * DD), lambda i: (0, 0)),
            pl.BlockSpec((DD, 2 * DD), lambda i: (0, 0)),
            pl.BlockSpec((1, 2 * DD), lambda i: (0, 0)),
            pl.BlockSpec((2 * DD, DD), lambda i: (0, 0)),
            pl.BlockSpec((1, DD), lambda i: (0, 0)),
            pl.BlockSpec((8, DD), lambda i: (0, 0)),
        ],
        out_specs=pl.BlockSpec((ROWS, DD), lambda i: (i, 0)),
        out_shape=jax.ShapeDtypeStruct((NB * NN, DD), jnp.float32),
    )(hid, w1a, w1b, b1, w2p, b2p, q)


def _gath_body(sch, ti, out, ti_v, rows, sem):
    c = lax.axis_index("c")
    s = lax.axis_index("s")

    @pl.when((c == 0) & (s == 0))
    def _():
        pltpu.sync_copy(ti, ti_v)
        pltpu.async_copy(sch.at[ti_v], rows, sem).wait()
        pltpu.sync_copy(rows, out)


@functools.lru_cache(maxsize=None)
def _get_gath_kernel():
    return pl.kernel(
        _gath_body,
        out_type=jax.ShapeDtypeStruct((80, DD), jnp.float32),
        mesh=plsc.VectorSubcoreMesh(core_axis_name="c", subcore_axis_name="s", num_cores=2, num_subcores=16),
        scratch_types=[
            pltpu.VMEM((80,), jnp.int32),
            pltpu.VMEM((80, DD), jnp.float32),
            pltpu.SemaphoreType.DMA,
        ],
    )


def kernel(edge_index, edge_type, h_index, t_index, r_index, query, rel_w,
           lin_w, lin_b, ln_scale, ln_bias, mlp_w1, mlp_b1, mlp_w2, mlp_b2):
    f32 = jnp.float32
    i32 = jnp.int32
    src = edge_index[0]
    dstv = edge_index[1]
    padn = EP - NE
    src_p = jnp.concatenate([src, jnp.zeros((padn,), i32)])
    srcg = jnp.concatenate([src_p, src_p + NN])
    dst_p = jnp.concatenate([dstv, jnp.full((padn,), NN, i32)])
    et_p = jnp.concatenate([edge_type, jnp.zeros((padn,), i32)])
    rels = [jnp.pad(rel_w[0], ((0, 2), (0, 0))),
            jnp.pad(rel_w[1], ((0, 2), (0, 0)))]
    hidx2 = h_index.reshape(1, NB)
    oh = (jnp.arange(104, dtype=i32)[None, :]
          == jnp.pad(r_index, (0, 6), constant_values=-1)[:, None]).astype(f32)
    qpad = jnp.pad(query, ((0, 2), (0, 0)))

    bound, q8 = _bound_call(hidx2, oh, qpad)
    hid = bound
    for i in range(2):
        aggr = _get_msg_kernel()(hid, srcg, dst_p, et_p, rels[i])
        w1 = lin_w[i, :DD, :]
        w2 = lin_w[i, DD:, :]
        hid = _dense_call(hid, aggr, w1, w2, lin_b[i].reshape(1, DD),
                          ln_scale[i].reshape(1, DD), ln_bias[i].reshape(1, DD),
                          hidx2, q8)

    w1a = mlp_w1[:DD, :]
    w1b = mlp_w1[DD:, :]
    w2p = jnp.pad(mlp_w2, ((0, 0), (0, DD - 1)))
    b2p = jnp.pad(mlp_b2.reshape(1, 1), ((0, 0), (0, DD - 1)))
    scoretab = _score_call(hid, w1a, w1b, mlp_b1.reshape(1, 2 * DD), w2p, b2p,
                           q8)

    tflat = (t_index.astype(i32)
             + NN * jnp.arange(NB, dtype=i32)[:, None]).reshape(-1)
    tflat = jnp.concatenate([tflat, jnp.zeros((80 - NB * NTAIL,), i32)])
    out80 = _get_gath_kernel()(scoretab, tflat)
    return out80[:NB * NTAIL, 0].reshape(NB, NTAIL)


# speedup vs baseline: 33.6760x; 1.2732x over previous
import jax
import jax.numpy as jnp
from jax import lax
from jax.experimental import pallas as pl
from jax.experimental.pallas import tpu as pltpu
from jax.experimental.pallas import tpu_sc as plsc

NN = 10000
NE = 160000
DD = 128
NB = 2
NTAIL = 34
NREL = 102
EPS = 1e-5

NSUB = 16
EPT = 10240
EP = EPT * NSUB
CH = 128
NCHUNK = EPT // CH
ACC_ROWS = 10000

ROWS = 1000
NBLK = NN // ROWS
GRID = NB * NBLK


def _msg_body(hid, srcg, dst, et, rel, agg,
              acc, rel_sp, srcA, srcB, dstA, dstB, etA, etB,
              hrowA, hrowB, rrow, semA, semB):
    c = lax.axis_index("c")
    s = lax.axis_index("s")
    z16 = jnp.zeros((16,), jnp.float32)

    def zrow(j, carry):
        for k in range(8):
            hrowA[j, pl.ds(k * 16, 16)] = z16
        return carry
    lax.fori_loop(0, CH, zrow, 0)

    zb = s * 625
    for k in range(4):
        pltpu.sync_copy(hrowA, acc.at[pl.ds(zb + k * 128, 128)])
    pltpu.sync_copy(hrowA.at[pl.ds(0, 113)], acc.at[pl.ds(zb + 512, 113)])

    @pl.when(s == 0)
    def _():
        pltpu.sync_copy(rel, hrowA.at[pl.ds(0, 104)])
        pltpu.sync_copy(hrowA.at[pl.ds(0, 104)], rel_sp)

    plsc.subcore_barrier()

    ebase = s * EPT

    def _load_idx(i, sv, dv, ev):
        eoff = ebase + i * CH
        pltpu.sync_copy(srcg.at[pl.ds(c * EP + eoff, CH)], sv)
        pltpu.sync_copy(dst.at[pl.ds(eoff, CH)], dv)
        pltpu.sync_copy(et.at[pl.ds(eoff, CH)], ev)

    def _mul(hb):
        def mulrow(j, cc):
            for k in range(8):
                sl = pl.ds(k * 16, 16)
                hb[j, sl] = hb[j, sl] * rrow[j, sl]
            return cc
        lax.fori_loop(0, CH, mulrow, 0)

    _load_idx(0, srcA, dstA, etA)
    pltpu.async_copy(hid.at[srcA], hrowA, semA)

    def pair(j, carry):
        _load_idx(2 * j + 1, srcB, dstB, etB)
        pltpu.async_copy(hid.at[srcB], hrowB, semB)

        pltpu.make_async_copy(hid.at[srcA], hrowA, semA).wait()
        pltpu.sync_copy(rel_sp.at[etA], rrow)
        _mul(hrowA)
        pltpu.sync_copy(hrowA, acc.at[dstA], add=True)

        @pl.when(j < NCHUNK // 2 - 1)
        def _():
            _load_idx(2 * j + 2, srcA, dstA, etA)
            pltpu.async_copy(hid.at[srcA], hrowA, semA)

        pltpu.make_async_copy(hid.at[srcB], hrowB, semB).wait()
        pltpu.sync_copy(rel_sp.at[etB], rrow)
        _mul(hrowB)
        pltpu.sync_copy(hrowB, acc.at[dstB], add=True)
        return carry
    lax.fori_loop(0, NCHUNK // 2, pair, 0)

    plsc.subcore_barrier()

    ob = s * 632

    @pl.when(s < 15)
    def _():
        for k in range(4):
            pltpu.sync_copy(acc.at[pl.ds(ob + k * 128, 128)], hrowA)
            pltpu.sync_copy(hrowA, agg.at[pl.ds(c * NN + ob + k * 128, 128)])
        pltpu.sync_copy(acc.at[pl.ds(ob + 512, 120)], hrowA.at[pl.ds(0, 120)])
        pltpu.sync_copy(hrowA.at[pl.ds(0, 120)],
                        agg.at[pl.ds(c * NN + ob + 512, 120)])

    @pl.when(s == 15)
    def _():
        for k in range(4):
            pltpu.sync_copy(acc.at[pl.ds(ob + k * 128, 128)], hrowA)
            pltpu.sync_copy(hrowA, agg.at[pl.ds(c * NN + ob + k * 128, 128)])
        pltpu.sync_copy(acc.at[pl.ds(ob + 512, 8)], hrowA.at[pl.ds(0, 8)])
        pltpu.sync_copy(hrowA.at[pl.ds(0, 8)],
                        agg.at[pl.ds(c * NN + ob + 512, 8)])


import functools


@functools.lru_cache(maxsize=None)
def _get_msg_kernel():
    return pl.kernel(
    _msg_body,
    out_type=jax.ShapeDtypeStruct((NB * NN, DD), jnp.float32),
    mesh=plsc.VectorSubcoreMesh(core_axis_name="c", subcore_axis_name="s", num_cores=2, num_subcores=16),
    scratch_types=[
        pltpu.VMEM_SHARED((ACC_ROWS, DD), jnp.float32),
        pltpu.VMEM_SHARED((104, DD), jnp.float32),
        pltpu.VMEM((CH,), jnp.int32),
        pltpu.VMEM((CH,), jnp.int32),
        pltpu.VMEM((CH,), jnp.int32),
        pltpu.VMEM((CH,), jnp.int32),
        pltpu.VMEM((CH,), jnp.int32),
        pltpu.VMEM((CH,), jnp.int32),
        pltpu.VMEM((CH, DD), jnp.float32),
        pltpu.VMEM((CH, DD), jnp.float32),
        pltpu.VMEM((CH, DD), jnp.float32),
        pltpu.SemaphoreType.DMA,
        pltpu.SemaphoreType.DMA,
    ],
    )


def _bound_body(hidx_ref, oh_ref, query_ref, bout_ref, qout_ref):
    i = pl.program_id(0)
    b = i // NBLK
    base_n = (i % NBLK) * ROWS
    q = jnp.dot(oh_ref[...], query_ref[...], preferred_element_type=jnp.float32)
    qout_ref[...] = q
    hi = jnp.where(b == 0, hidx_ref[0, 0], hidx_ref[0, 1])
    mask = (lax.broadcasted_iota(jnp.int32, (ROWS, 1), 0) + base_n == hi
            ).astype(jnp.float32)
    qsel = jnp.where(b == 0, q[0:1, :], q[1:2, :])
    bout_ref[...] = mask * qsel


def _bound_call(hidx, oh, qpad):
    return pl.pallas_call(
        _bound_body,
        grid=(GRID,),
        in_specs=[
            pl.BlockSpec(memory_space=pltpu.SMEM),
            pl.BlockSpec((8, 104), lambda i: (0, 0)),
            pl.BlockSpec((104, DD), lambda i: (0, 0)),
        ],
        out_specs=[
            pl.BlockSpec((ROWS, DD), lambda i: (i, 0)),
            pl.BlockSpec((8, DD), lambda i: (0, 0)),
        ],
        out_shape=[
            jax.ShapeDtypeStruct((NB * NN, DD), jnp.float32),
            jax.ShapeDtypeStruct((8, DD), jnp.float32),
        ],
    )(hidx, oh, qpad)


def _dense_body(hidx_ref, hid_ref, agg_ref, w1_ref, w2_ref, bb_ref, g_ref,
                be_ref, q_ref, out_ref):
    i = pl.program_id(0)
    b = i // NBLK
    base_n = (i % NBLK) * ROWS
    h = hid_ref[...]
    a = agg_ref[...]
    hi = jnp.where(b == 0, hidx_ref[0, 0], hidx_ref[0, 1])
    mask = (lax.broadcasted_iota(jnp.int32, (ROWS, 1), 0) + base_n == hi
            ).astype(jnp.float32)
    qsel = jnp.where(b == 0, q_ref[0:1, :], q_ref[1:2, :])
    a = a + mask * qsel
    out = (jnp.dot(h, w1_ref[...], preferred_element_type=jnp.float32)
           + jnp.dot(a, w2_ref[...], preferred_element_type=jnp.float32)
           + bb_ref[...])
    mean = jnp.mean(out, axis=-1, keepdims=True)
    var = jnp.mean((out - mean) * (out - mean), axis=-1, keepdims=True)
    out = (out - mean) / jnp.sqrt(var + EPS) * g_ref[...] + be_ref[...]
    out_ref[...] = jnp.maximum(out, 0.0) + h


def _dense_call(hid, aggr, w1, w2, bb, g, be, hidx, q):
    return pl.pallas_call(
        _dense_body,
        grid=(GRID,),
        in_specs=[
            pl.BlockSpec(memory_space=pltpu.SMEM),
            pl.BlockSpec((ROWS, DD), lambda i: (i, 0)),
            pl.BlockSpec((ROWS, DD), lambda i: (i, 0)),
            pl.BlockSpec((DD, DD), lambda i: (0, 0)),
            pl.BlockSpec((DD, DD), lambda i: (0, 0)),
            pl.BlockSpec((1, DD), lambda i: (0, 0)),
            pl.BlockSpec((1, DD), lambda i: (0, 0)),
            pl.BlockSpec((1, DD), lambda i: (0, 0)),
            pl.BlockSpec((8, DD), lambda i: (0, 0)),
        ],
        out_specs=pl.BlockSpec((ROWS, DD), lambda i: (i, 0)),
        out_shape=jax.ShapeDtypeStruct((NB * NN, DD), jnp.float32),
    )(hidx, hid, aggr, w1, w2, bb, g, be, q)


def _score_body(hid_ref, w1a_ref, w1b_ref, b1_ref, w2_ref, b2_ref, q_ref,
                out_ref):
    i = pl.program_id(0)
    b = i // NBLK
    h = hid_ref[...]
    qsel = jnp.where(b == 0, q_ref[0:1, :], q_ref[1:2, :])
    pre = (jnp.dot(qsel, w1b_ref[...], preferred_element_type=jnp.float32)
           + b1_ref[...])
    h1 = jnp.maximum(
        jnp.dot(h, w1a_ref[...], preferred_element_type=jnp.float32) + pre, 0.0)
    out_ref[...] = (jnp.dot(h1, w2_ref[...], preferred_element_type=jnp.float32)
                    + b2_ref[...])


def _score_call(hid, w1a, w1b, b1, w2p, b2p, q):
    return pl.pallas_call(
        _score_body,
        grid=(GRID,),
        in_specs=[
            pl.BlockSpec((ROWS, DD), lambda i: (i, 0)),
            pl.BlockSpec((DD, 2 * DD), lambda i: (0, 0)),
            pl.BlockSpec((DD, 2 * DD), lambda i: (0, 0)),
            pl.BlockSpec((1, 2 * DD), lambda i: (0, 0)),
            pl.BlockSpec((2 * DD, DD), lambda i: (0, 0)),
            pl.BlockSpec((1, DD), lambda i: (0, 0)),
            pl.BlockSpec((8, DD), lambda i: (0, 0)),
        ],
        out_specs=pl.BlockSpec((ROWS, DD), lambda i: (i, 0)),
        out_shape=jax.ShapeDtypeStruct((NB * NN, DD), jnp.float32),
    )(hid, w1a, w1b, b1, w2p, b2p, q)


def _gath_body(sch, ti, out, ti_v, rows, sem):
    c = lax.axis_index("c")
    s = lax.axis_index("s")

    @pl.when((c == 0) & (s == 0))
    def _():
        pltpu.sync_copy(ti, ti_v)
        pltpu.async_copy(sch.at[ti_v], rows, sem).wait()
        pltpu.sync_copy(rows, out)


@functools.lru_cache(maxsize=None)
def _get_gath_kernel():
    return pl.kernel(
        _gath_body,
        out_type=jax.ShapeDtypeStruct((80, DD), jnp.float32),
        mesh=plsc.VectorSubcoreMesh(core_axis_name="c", subcore_axis_name="s", num_cores=2, num_subcores=16),
        scratch_types=[
            pltpu.VMEM((80,), jnp.int32),
            pltpu.VMEM((80, DD), jnp.float32),
            pltpu.SemaphoreType.DMA,
        ],
    )


def kernel(edge_index, edge_type, h_index, t_index, r_index, query, rel_w,
           lin_w, lin_b, ln_scale, ln_bias, mlp_w1, mlp_b1, mlp_w2, mlp_b2):
    f32 = jnp.float32
    i32 = jnp.int32
    src = edge_index[0]
    dstv = edge_index[1]
    padn = EP - NE
    src_p = jnp.concatenate([src, jnp.zeros((padn,), i32)])
    srcg = jnp.concatenate([src_p, src_p + NN])
    dst_p = jnp.concatenate([dstv, jnp.zeros((padn,), i32)])
    et_p = jnp.concatenate([edge_type, jnp.full((padn,), NREL, i32)])
    rels = [jnp.pad(rel_w[0], ((0, 2), (0, 0))),
            jnp.pad(rel_w[1], ((0, 2), (0, 0)))]
    hidx2 = h_index.reshape(1, NB)
    oh = (jnp.arange(104, dtype=i32)[None, :]
          == jnp.pad(r_index, (0, 6), constant_values=-1)[:, None]).astype(f32)
    qpad = jnp.pad(query, ((0, 2), (0, 0)))

    bound, q8 = _bound_call(hidx2, oh, qpad)
    hid = bound
    for i in range(2):
        aggr = _get_msg_kernel()(hid, srcg, dst_p, et_p, rels[i])
        w1 = lin_w[i, :DD, :]
        w2 = lin_w[i, DD:, :]
        hid = _dense_call(hid, aggr, w1, w2, lin_b[i].reshape(1, DD),
                          ln_scale[i].reshape(1, DD), ln_bias[i].reshape(1, DD),
                          hidx2, q8)

    w1a = mlp_w1[:DD, :]
    w1b = mlp_w1[DD:, :]
    w2p = jnp.pad(mlp_w2, ((0, 0), (0, DD - 1)))
    b2p = jnp.pad(mlp_b2.reshape(1, 1), ((0, 0), (0, DD - 1)))
    scoretab = _score_call(hid, w1a, w1b, mlp_b1.reshape(1, 2 * DD), w2p, b2p,
                           q8)

    tflat = (t_index.astype(i32)
             + NN * jnp.arange(NB, dtype=i32)[:, None]).reshape(-1)
    tflat = jnp.concatenate([tflat, jnp.zeros((80 - NB * NTAIL,), i32)])
    out80 = _get_gath_kernel()(scoretab, tflat)
    return out80[:NB * NTAIL, 0].reshape(NB, NTAIL)
